# baseline (speedup 1.0000x reference)
import jax
import jax.numpy as jnp
from jax import lax
from jax.experimental import pallas as pl
from jax.experimental.pallas import tpu as pltpu
from jax.experimental.pallas import tpu_sc as plsc

_BATCH = 16384
_PERIOD = 50
_HID = 16

_NC = 2
_NS = 16
_NW = _NC * _NS
_N = _BATCH * _HID
_SLICE = _N // _NW


_HALF = _PERIOD // 2
_SPAN = _HALF * _N // _NS
_UNIT = 16384
_NUNIT = _SPAN // _UNIT
_LOAD = _N // _NS


def _body(w_hbm, out_hbm, w_sh, sem):
    c = lax.axis_index("c")
    s = lax.axis_index("s")
    pltpu.sync_copy(w_hbm.at[pl.ds(s * _LOAD, _LOAD)], w_sh.at[pl.ds(s * _LOAD, _LOAD)])
    plsc.subcore_barrier()
    span_start = c * _HALF * _N + s * _SPAN
    rel_start = s * _SPAN

    def issue(i, carry):
        rel = rel_start + i * _UNIT
        src = lax.rem(rel, _N)
        pltpu.async_copy(
            w_sh.at[pl.ds(src, _UNIT)],
            out_hbm.at[pl.ds(span_start + i * _UNIT, _UNIT)],
            sem,
        )
        return carry

    lax.fori_loop(0, _NUNIT, issue, 0)
    pltpu.make_async_copy(
        out_hbm.at[pl.ds(span_start, _SPAN)],
        out_hbm.at[pl.ds(span_start, _SPAN)],
        sem,
    ).wait()


def kernel(x, W):
    mesh = plsc.VectorSubcoreMesh(core_axis_name="c", subcore_axis_name="s")
    k = pl.kernel(
        _body,
        out_type=jax.ShapeDtypeStruct((_PERIOD * _N,), jnp.float32),
        mesh=mesh,
        scratch_types=[
            pltpu.VMEM_SHARED((_N,), jnp.float32),
            pltpu.SemaphoreType.DMA,
        ],
    )
    w_flat = W.reshape(128, 128, 2, 8).transpose(2, 0, 3, 1).reshape(-1)
    flat = k(w_flat)
    return (
        flat.reshape(_PERIOD, 2, 128, 8, 128)
        .transpose(2, 4, 0, 1, 3)
        .reshape(_BATCH, _PERIOD, _HID)
    )


# speedup vs baseline: 1.3035x; 1.3035x over previous
import jax
import jax.numpy as jnp
from jax import lax
from jax.experimental import pallas as pl
from jax.experimental.pallas import tpu as pltpu
from jax.experimental.pallas import tpu_sc as plsc

_BATCH = 16384
_PERIOD = 50
_HID = 16

_NC = 2
_NS = 16
_NW = _NC * _NS
_N = _BATCH * _HID
_SLICE = _N // _NW


_HALF = _PERIOD // 2
_PIECE = _N // _NS


def _body(w_hbm, out_hbm, w_v, sem):
    c = lax.axis_index("c")
    s = lax.axis_index("s")
    pltpu.sync_copy(w_hbm.at[pl.ds(s * _PIECE, _PIECE)], w_v)
    first = c * _HALF

    def issue(i, carry):
        pltpu.async_copy(
            w_v,
            out_hbm.at[pl.ds((first + i) * _N + s * _PIECE, _PIECE)],
            sem,
        )
        return carry

    lax.fori_loop(0, _HALF, issue, 0)
    pltpu.make_async_copy(
        out_hbm.at[pl.ds(0, _HALF * _PIECE)],
        out_hbm.at[pl.ds(0, _HALF * _PIECE)],
        sem,
    ).wait()


def kernel(x, W):
    mesh = plsc.VectorSubcoreMesh(core_axis_name="c", subcore_axis_name="s")
    k = pl.kernel(
        _body,
        out_type=jax.ShapeDtypeStruct((_PERIOD * _N,), jnp.float32),
        mesh=mesh,
        scratch_types=[
            pltpu.VMEM((_PIECE,), jnp.float32),
            pltpu.SemaphoreType.DMA,
        ],
    )
    w_flat = W.reshape(128, 128, 2, 8).transpose(2, 0, 3, 1).reshape(-1)
    flat = k(w_flat)
    return (
        flat.reshape(_PERIOD, 2, 128, 8, 128)
        .transpose(2, 4, 0, 1, 3)
        .reshape(_BATCH, _PERIOD, _HID)
    )
